# Initial kernel scaffold; baseline (speedup 1.0000x reference)
import jax
import jax.numpy as jnp
from jax.experimental import pallas as pl


def kernel(x, exp_emb_table, exp_bias_table, gcn_w, gcn_b, lin_w, lin_b, edge_index, batch):
    raise NotImplementedError("write your pallas kernel here")


# speedup vs baseline: 108.0497x; 108.0497x over previous
import functools

import jax
import jax.numpy as jnp
from jax import lax
from jax.experimental import pallas as pl
from jax.experimental.pallas import tpu as pltpu
from jax.experimental.pallas import tpu_sc as plsc

NUM_GENES = 10000
NUM_CLASS = 32
BATCH_SIZE = 10
N = NUM_GENES * BATCH_SIZE
E = 1600000
K = 5000

N_PAD = 100096
SLICE = N_PAD // 16
EROWS = 12512
ROWS_W = EROWS // 32
CHUNK = 23
NCHUNK = ROWS_W // CHUNK

INT_MIN = -2**31


def _zero_fill(buf, n):
    zero = jnp.zeros((16,), jnp.float32)

    def body(i, _):
        buf[pl.ds(i * 16, 16)] = zero
        return 0

    lax.fori_loop(0, n // 16, body, 0)


CHUNK_E = CHUNK * 128


def _sc_hist(dst_hbm, out0_hbm, out1_hbm, idx_v, ones_v, bounce_v, acc_sh):
    c = lax.axis_index("c")
    s = lax.axis_index("s")
    wid = s * 2 + c

    one = jnp.ones((16,), jnp.float32)

    def fill(r, _):
        ones_v[pl.ds(r * 16, 16)] = one
        return 0

    lax.fori_loop(0, CHUNK_E // 16, fill, 0)

    _zero_fill(bounce_v, SLICE)
    pltpu.sync_copy(bounce_v, acc_sh.at[pl.ds(s * SLICE, SLICE)])
    plsc.subcore_barrier()

    def body(i, _):
        e0 = (wid * ROWS_W + i * CHUNK) * 128
        pltpu.sync_copy(dst_hbm.at[pl.ds(e0, CHUNK_E)], idx_v)
        pltpu.sync_copy(ones_v, acc_sh.at[idx_v], add=True)
        return 0

    lax.fori_loop(0, NCHUNK, body, 0)
    plsc.subcore_barrier()

    pltpu.sync_copy(acc_sh.at[pl.ds(s * SLICE, SLICE)], bounce_v)

    @pl.when(c == 0)
    def _():
        pltpu.sync_copy(bounce_v, out0_hbm.at[pl.ds(s * SLICE, SLICE)])

    @pl.when(c == 1)
    def _():
        pltpu.sync_copy(bounce_v, out1_hbm.at[pl.ds(s * SLICE, SLICE)])


def _sc_msg(src_hbm, dst_hbm, g_hbm, out0_hbm, out1_hbm, sidx_v, didx_v,
            vals_v, bounce_v, acc_sh, sem):
    c = lax.axis_index("c")
    s = lax.axis_index("s")
    wid = s * 2 + c

    _zero_fill(bounce_v, SLICE)
    pltpu.sync_copy(bounce_v, acc_sh.at[pl.ds(s * SLICE, SLICE)])
    plsc.subcore_barrier()

    def body(i, _):
        e0 = (wid * ROWS_W + i * CHUNK) * 128
        pltpu.sync_copy(src_hbm.at[pl.ds(e0, CHUNK_E)], sidx_v)
        pltpu.sync_copy(dst_hbm.at[pl.ds(e0, CHUNK_E)], didx_v)
        pltpu.async_copy(g_hbm.at[sidx_v], vals_v, sem).wait()
        pltpu.sync_copy(vals_v, acc_sh.at[didx_v], add=True)
        return 0

    lax.fori_loop(0, NCHUNK, body, 0)
    plsc.subcore_barrier()

    pltpu.sync_copy(acc_sh.at[pl.ds(s * SLICE, SLICE)], bounce_v)

    @pl.when(c == 0)
    def _():
        pltpu.sync_copy(bounce_v, out0_hbm.at[pl.ds(s * SLICE, SLICE)])

    @pl.when(c == 1)
    def _():
        pltpu.sync_copy(bounce_v, out1_hbm.at[pl.ds(s * SLICE, SLICE)])


def _deg_call():
    mesh = plsc.VectorSubcoreMesh(core_axis_name="c", subcore_axis_name="s")
    return functools.partial(
        pl.kernel, _sc_hist, mesh=mesh,
        out_type=[jax.ShapeDtypeStruct((N_PAD,), jnp.float32),
                  jax.ShapeDtypeStruct((N_PAD,), jnp.float32)],
        scratch_types=[
            pltpu.VMEM((CHUNK_E,), jnp.int32),
            pltpu.VMEM((CHUNK_E,), jnp.float32),
            pltpu.VMEM((SLICE,), jnp.float32),
            pltpu.VMEM_SHARED((N_PAD,), jnp.float32),
        ],
    )()


def _msg_call():
    mesh = plsc.VectorSubcoreMesh(core_axis_name="c", subcore_axis_name="s")
    return functools.partial(
        pl.kernel, _sc_msg, mesh=mesh,
        out_type=[jax.ShapeDtypeStruct((N_PAD,), jnp.float32),
                  jax.ShapeDtypeStruct((N_PAD,), jnp.float32)],
        scratch_types=[
            pltpu.VMEM((CHUNK_E,), jnp.int32),
            pltpu.VMEM((CHUNK_E,), jnp.int32),
            pltpu.VMEM((CHUNK_E,), jnp.float32),
            pltpu.VMEM((SLICE,), jnp.float32),
            pltpu.VMEM_SHARED((N_PAD,), jnp.float32),
            pltpu.SemaphoreType.DMA,
        ],
    )()


def _tc_prep(x_ref, emb_ref, bias_ref, w_ref, degp_ref, dis_ref, g_ref):
    xb = x_ref[0]
    h = emb_ref[...] * xb + bias_ref[...]
    hw = jnp.sum(h * w_ref[...], axis=1)
    deg = 1.0 + degp_ref[0, 0] + degp_ref[0, 1]
    dis = lax.rsqrt(deg)
    dis_ref[0, 0, :] = dis
    g_ref[0, 0, :] = hw * dis


def _tc_score(dis_ref, g_ref, tp_ref, gcnb_ref, wgt_ref):
    t = tp_ref[0] + tp_ref[1]
    score = dis_ref[...] * (t + g_ref[...]) + gcnb_ref[0, 0]
    bits = lax.bitcast_convert_type(score, jnp.int32)
    skey = bits ^ ((bits >> 31) & jnp.int32(0x7FFFFFFF))

    int_min = jnp.int32(INT_MIN)
    thr_u = jnp.zeros((BATCH_SIZE, 1), jnp.int32)
    for b in range(31, -1, -1):
        bit = int_min if b == 31 else jnp.int32(1 << b)
        cand = thr_u | bit
        cand_s = cand ^ int_min
        cnt = jnp.sum((skey >= cand_s).astype(jnp.int32), axis=1,
                      keepdims=True)
        thr_u = jnp.where(cnt >= K, cand, thr_u)
    thr_s = thr_u ^ int_min

    wgt_ref[...] = jnp.where(skey >= thr_s, jnp.tanh(score), 0.0)


def _tc_pool(x_ref, emb_ref, bias_ref, wgt_ref, linw_ref, linb_ref, out_ref):
    h = emb_ref[...] * x_ref[0] + bias_ref[...]
    wcol = wgt_ref[0, 0].reshape(NUM_GENES, 1)
    pooled = jnp.sum(h * wcol, axis=0) * jnp.float32(1.0 / K)
    row = jnp.sum(linw_ref[...] * pooled[None, :], axis=1)
    out_ref[0, 0, :] = row + linb_ref[0]


def _prep_call(x3, emb, bias, w_row, degp3, *, interpret=False):
    return pl.pallas_call(
        _tc_prep,
        grid=(BATCH_SIZE,),
        in_specs=[
            pl.BlockSpec((1, NUM_GENES, NUM_CLASS), lambda b: (b, 0, 0)),
            pl.BlockSpec((NUM_GENES, NUM_CLASS), lambda b: (0, 0)),
            pl.BlockSpec((NUM_GENES, 1), lambda b: (0, 0)),
            pl.BlockSpec((1, NUM_CLASS), lambda b: (0, 0)),
            pl.BlockSpec((1, 2, NUM_GENES), lambda b: (b, 0, 0)),
        ],
        out_specs=[
            pl.BlockSpec((1, 1, NUM_GENES), lambda b: (b, 0, 0)),
            pl.BlockSpec((1, 1, NUM_GENES), lambda b: (b, 0, 0)),
        ],
        out_shape=[
            jax.ShapeDtypeStruct((BATCH_SIZE, 1, NUM_GENES), jnp.float32),
            jax.ShapeDtypeStruct((BATCH_SIZE, 1, NUM_GENES), jnp.float32),
        ],
        interpret=interpret,
    )(x3, emb, bias, w_row, degp3)


def _final_call(x3, emb, bias, dis2, g2, tp3, gcnb, linw, linb, *,
                interpret=False):
    wgt = pl.pallas_call(
        _tc_score,
        out_shape=jax.ShapeDtypeStruct((BATCH_SIZE, NUM_GENES), jnp.float32),
        interpret=interpret,
    )(dis2, g2, tp3, gcnb)
    wgt3 = wgt.reshape(BATCH_SIZE, 1, NUM_GENES)
    out = pl.pallas_call(
        _tc_pool,
        grid=(BATCH_SIZE,),
        in_specs=[
            pl.BlockSpec((1, NUM_GENES, NUM_CLASS), lambda b: (b, 0, 0)),
            pl.BlockSpec((NUM_GENES, NUM_CLASS), lambda b: (0, 0)),
            pl.BlockSpec((NUM_GENES, 1), lambda b: (0, 0)),
            pl.BlockSpec((1, 1, NUM_GENES), lambda b: (b, 0, 0)),
            pl.BlockSpec((NUM_CLASS, NUM_CLASS), lambda b: (0, 0)),
            pl.BlockSpec((1, NUM_CLASS), lambda b: (0, 0)),
        ],
        out_specs=pl.BlockSpec((1, 1, NUM_CLASS), lambda b: (b, 0, 0)),
        out_shape=jax.ShapeDtypeStruct((BATCH_SIZE, 1, NUM_CLASS),
                                       jnp.float32),
        interpret=interpret,
    )(x3, emb, bias, wgt3, linw, linb)
    return out.reshape(BATCH_SIZE, NUM_CLASS)


def kernel(x, exp_emb_table, exp_bias_table, gcn_w, gcn_b, lin_w, lin_b,
           edge_index, batch):
    src = edge_index[0]
    dst = edge_index[1]
    pad = EROWS * 128 - E
    dstp = jnp.concatenate([dst, jnp.full((pad,), N_PAD - 1, jnp.int32)])
    srcp = jnp.concatenate([src, jnp.zeros((pad,), jnp.int32)])

    deg0, deg1 = _deg_call()(dstp)

    x3 = x.reshape(BATCH_SIZE, NUM_GENES, NUM_CLASS)
    degp3 = jnp.stack([deg0[:N], deg1[:N]]).reshape(
        2, BATCH_SIZE, NUM_GENES).transpose(1, 0, 2)
    dis2, g2 = _prep_call(x3, exp_emb_table, exp_bias_table,
                          gcn_w.reshape(1, NUM_CLASS), degp3)
    dis2 = dis2.reshape(BATCH_SIZE, NUM_GENES)
    g2 = g2.reshape(BATCH_SIZE, NUM_GENES)

    g_flat = jnp.concatenate(
        [g2.reshape(N), jnp.zeros((N_PAD - N,), jnp.float32)])
    t0, t1 = _msg_call()(srcp, dstp, g_flat)
    tp3 = jnp.stack([t0[:N], t1[:N]]).reshape(2, BATCH_SIZE, NUM_GENES)

    out = _final_call(x3, exp_emb_table, exp_bias_table, dis2, g2, tp3,
                      gcn_b.reshape(1, 1), lin_w, lin_b.reshape(1, NUM_CLASS))
    return out


# speedup vs baseline: 131.1166x; 1.2135x over previous
import functools

import jax
import jax.numpy as jnp
from jax import lax
from jax.experimental import pallas as pl
from jax.experimental.pallas import tpu as pltpu
from jax.experimental.pallas import tpu_sc as plsc

NUM_GENES = 10000
NUM_CLASS = 32
BATCH_SIZE = 10
N = NUM_GENES * BATCH_SIZE
E = 1600000
K = 5000

N_PAD = 100096
SLICE = N_PAD // 16
E_PAD = 1601536
EPW = E_PAD // 32
CH = 2944
ITER = EPW // CH

INT_MIN = -2**31


def _zero_fill(buf, n):
    zero = jnp.zeros((16,), jnp.float32)

    def body(i, _):
        buf[pl.ds(i * 16, 16)] = zero
        return 0

    lax.fori_loop(0, n // 16, body, 0)


def _sc_hist(dst_hbm, out0_hbm, out1_hbm, idx0_v, idx1_v, ones_v, bounce_v,
             acc_sh, sem0, sem1):
    c = lax.axis_index("c")
    s = lax.axis_index("s")
    wid = s * 2 + c
    base = wid * EPW

    one = jnp.ones((16,), jnp.float32)

    def fill(i, _):
        ones_v[pl.ds(i * 16, 16)] = one
        return 0

    lax.fori_loop(0, CH // 16, fill, 0)

    _zero_fill(bounce_v, SLICE)
    pltpu.sync_copy(bounce_v, acc_sh.at[pl.ds(s * SLICE, SLICE)])
    plsc.subcore_barrier()

    idx = [idx0_v, idx1_v]
    sems = [sem0, sem1]
    loads = [None] * ITER
    for i in range(2):
        loads[i] = pltpu.async_copy(
            dst_hbm.at[pl.ds(base + i * CH, CH)], idx[i], sems[i])
    for i in range(ITER):
        b = i % 2
        loads[i].wait()
        pltpu.sync_copy(ones_v, acc_sh.at[idx[b]], add=True)
        if i + 2 < ITER:
            loads[i + 2] = pltpu.async_copy(
                dst_hbm.at[pl.ds(base + (i + 2) * CH, CH)], idx[b], sems[b])
    plsc.subcore_barrier()

    pltpu.sync_copy(acc_sh.at[pl.ds(s * SLICE, SLICE)], bounce_v)

    @pl.when(c == 0)
    def _():
        pltpu.sync_copy(bounce_v, out0_hbm.at[pl.ds(s * SLICE, SLICE)])

    @pl.when(c == 1)
    def _():
        pltpu.sync_copy(bounce_v, out1_hbm.at[pl.ds(s * SLICE, SLICE)])


def _sc_msg(src_hbm, dst_hbm, g_hbm, out0_hbm, out1_hbm,
            sidx0_v, sidx1_v, didx0_v, didx1_v, vals0_v, vals1_v,
            bounce_v, acc_sh, sems0, sems1, semd0, semd1, semg0, semg1):
    c = lax.axis_index("c")
    s = lax.axis_index("s")
    wid = s * 2 + c
    base = wid * EPW

    _zero_fill(bounce_v, SLICE)
    pltpu.sync_copy(bounce_v, acc_sh.at[pl.ds(s * SLICE, SLICE)])
    plsc.subcore_barrier()

    sidx = [sidx0_v, sidx1_v]
    didx = [didx0_v, didx1_v]
    vals = [vals0_v, vals1_v]
    sems = [sems0, sems1]
    semd = [semd0, semd1]
    semg = [semg0, semg1]
    sload = [None] * ITER
    dload = [None] * ITER
    gath = [None] * ITER

    for i in range(2):
        sload[i] = pltpu.async_copy(
            src_hbm.at[pl.ds(base + i * CH, CH)], sidx[i], sems[i])
        dload[i] = pltpu.async_copy(
            dst_hbm.at[pl.ds(base + i * CH, CH)], didx[i], semd[i])
    sload[0].wait()
    gath[0] = pltpu.async_copy(g_hbm.at[sidx[0]], vals[0], semg[0])
    if ITER > 1:
        sload[1].wait()
        gath[1] = pltpu.async_copy(g_hbm.at[sidx[1]], vals[1], semg[1])
    for i in range(ITER):
        b = i % 2
        gath[i].wait()
        dload[i].wait()
        pltpu.sync_copy(vals[b], acc_sh.at[didx[b]], add=True)
        if i + 2 < ITER:
            sload[i + 2] = pltpu.async_copy(
                src_hbm.at[pl.ds(base + (i + 2) * CH, CH)], sidx[b], sems[b])
            dload[i + 2] = pltpu.async_copy(
                dst_hbm.at[pl.ds(base + (i + 2) * CH, CH)], didx[b], semd[b])
            sload[i + 2].wait()
            gath[i + 2] = pltpu.async_copy(g_hbm.at[sidx[b]], vals[b],
                                           semg[b])
    plsc.subcore_barrier()

    pltpu.sync_copy(acc_sh.at[pl.ds(s * SLICE, SLICE)], bounce_v)

    @pl.when(c == 0)
    def _():
        pltpu.sync_copy(bounce_v, out0_hbm.at[pl.ds(s * SLICE, SLICE)])

    @pl.when(c == 1)
    def _():
        pltpu.sync_copy(bounce_v, out1_hbm.at[pl.ds(s * SLICE, SLICE)])


def _deg_call():
    mesh = plsc.VectorSubcoreMesh(core_axis_name="c", subcore_axis_name="s")
    return functools.partial(
        pl.kernel, _sc_hist, mesh=mesh,
        out_type=[jax.ShapeDtypeStruct((N_PAD,), jnp.float32),
                  jax.ShapeDtypeStruct((N_PAD,), jnp.float32)],
        scratch_types=[
            pltpu.VMEM((CH,), jnp.int32),
            pltpu.VMEM((CH,), jnp.int32),
            pltpu.VMEM((CH,), jnp.float32),
            pltpu.VMEM((SLICE,), jnp.float32),
            pltpu.VMEM_SHARED((N_PAD,), jnp.float32),
            pltpu.SemaphoreType.DMA,
            pltpu.SemaphoreType.DMA,
        ],
    )()


def _msg_call():
    mesh = plsc.VectorSubcoreMesh(core_axis_name="c", subcore_axis_name="s")
    return functools.partial(
        pl.kernel, _sc_msg, mesh=mesh,
        out_type=[jax.ShapeDtypeStruct((N_PAD,), jnp.float32),
                  jax.ShapeDtypeStruct((N_PAD,), jnp.float32)],
        scratch_types=[
            pltpu.VMEM((CH,), jnp.int32),
            pltpu.VMEM((CH,), jnp.int32),
            pltpu.VMEM((CH,), jnp.int32),
            pltpu.VMEM((CH,), jnp.int32),
            pltpu.VMEM((CH,), jnp.float32),
            pltpu.VMEM((CH,), jnp.float32),
            pltpu.VMEM((SLICE,), jnp.float32),
            pltpu.VMEM_SHARED((N_PAD,), jnp.float32),
            pltpu.SemaphoreType.DMA,
            pltpu.SemaphoreType.DMA,
            pltpu.SemaphoreType.DMA,
            pltpu.SemaphoreType.DMA,
            pltpu.SemaphoreType.DMA,
            pltpu.SemaphoreType.DMA,
        ],
    )()


def _tc_hw(x_ref, emb_ref, bias_ref, w_ref, hw_ref):
    xb = x_ref[0]
    h = emb_ref[...] * xb + bias_ref[...]
    hw_ref[0, 0, :] = jnp.sum(h * w_ref[...], axis=1)


def _tc_mix(hw_ref, d0_ref, d1_ref, dis_ref, g_ref):
    deg = 1.0 + d0_ref[:, 0, :] + d1_ref[:, 0, :]
    dis = lax.rsqrt(deg)
    dis_ref[:, 0, :] = dis
    g_ref[:, 0, :] = hw_ref[:, 0, :] * dis


def _tc_score(dis_ref, g_ref, t0_ref, t1_ref, gcnb_ref, wgt_ref):
    t = t0_ref[:, 0, :] + t1_ref[:, 0, :]
    score = dis_ref[:, 0, :] * (t + g_ref[:, 0, :]) + gcnb_ref[0, 0]
    bits = lax.bitcast_convert_type(score, jnp.int32)
    skey = bits ^ ((bits >> 31) & jnp.int32(0x7FFFFFFF))

    int_min = jnp.int32(INT_MIN)
    thr_u = jnp.zeros((BATCH_SIZE, 1), jnp.int32)
    for b in range(31, -1, -1):
        bit = int_min if b == 31 else jnp.int32(1 << b)
        cand = thr_u | bit
        cand_s = cand ^ int_min
        cnt = jnp.sum((skey >= cand_s).astype(jnp.int32), axis=1,
                      keepdims=True)
        thr_u = jnp.where(cnt >= K, cand, thr_u)
    thr_s = thr_u ^ int_min

    wgt_ref[...] = jnp.where(skey >= thr_s, jnp.tanh(score), 0.0)


def _tc_pool(x_ref, emb_ref, bias_ref, wgt_ref, linw_ref, linb_ref, out_ref):
    h = emb_ref[...] * x_ref[0] + bias_ref[...]
    wcol = wgt_ref[0, 0].reshape(NUM_GENES, 1)
    pooled = jnp.sum(h * wcol, axis=0) * jnp.float32(1.0 / K)
    row = jnp.sum(linw_ref[...] * pooled[None, :], axis=1)
    out_ref[0, 0, :] = row + linb_ref[0]


def _hw_call(x3, emb, bias, w_row):
    return pl.pallas_call(
        _tc_hw,
        grid=(BATCH_SIZE,),
        in_specs=[
            pl.BlockSpec((1, NUM_GENES, NUM_CLASS), lambda b: (b, 0, 0)),
            pl.BlockSpec((NUM_GENES, NUM_CLASS), lambda b: (0, 0)),
            pl.BlockSpec((NUM_GENES, 1), lambda b: (0, 0)),
            pl.BlockSpec((1, NUM_CLASS), lambda b: (0, 0)),
        ],
        out_specs=pl.BlockSpec((1, 1, NUM_GENES), lambda b: (b, 0, 0)),
        out_shape=jax.ShapeDtypeStruct((BATCH_SIZE, 1, NUM_GENES),
                                       jnp.float32),
    )(x3, emb, bias, w_row)


def _mix_call(hw3, d03, d13):
    return pl.pallas_call(
        _tc_mix,
        out_shape=[
            jax.ShapeDtypeStruct((BATCH_SIZE, 1, NUM_GENES), jnp.float32),
            jax.ShapeDtypeStruct((BATCH_SIZE, 1, NUM_GENES), jnp.float32),
        ],
    )(hw3, d03, d13)


def _score_call(dis3, g3, t03, t13, gcnb):
    return pl.pallas_call(
        _tc_score,
        out_shape=jax.ShapeDtypeStruct((BATCH_SIZE, NUM_GENES), jnp.float32),
    )(dis3, g3, t03, t13, gcnb)


def _pool_call(x3, emb, bias, wgt3, linw, linb):
    return pl.pallas_call(
        _tc_pool,
        grid=(BATCH_SIZE,),
        in_specs=[
            pl.BlockSpec((1, NUM_GENES, NUM_CLASS), lambda b: (b, 0, 0)),
            pl.BlockSpec((NUM_GENES, NUM_CLASS), lambda b: (0, 0)),
            pl.BlockSpec((NUM_GENES, 1), lambda b: (0, 0)),
            pl.BlockSpec((1, 1, NUM_GENES), lambda b: (b, 0, 0)),
            pl.BlockSpec((NUM_CLASS, NUM_CLASS), lambda b: (0, 0)),
            pl.BlockSpec((1, NUM_CLASS), lambda b: (0, 0)),
        ],
        out_specs=pl.BlockSpec((1, 1, NUM_CLASS), lambda b: (b, 0, 0)),
        out_shape=jax.ShapeDtypeStruct((BATCH_SIZE, 1, NUM_CLASS),
                                       jnp.float32),
    )(x3, emb, bias, wgt3, linw, linb)


def _slice3(a):
    return a[:N].reshape(BATCH_SIZE, 1, NUM_GENES)


def kernel(x, exp_emb_table, exp_bias_table, gcn_w, gcn_b, lin_w, lin_b,
           edge_index, batch):
    pad = E_PAD - E
    src = jnp.concatenate([edge_index[0], jnp.zeros((pad,), jnp.int32)])
    dst = jnp.concatenate(
        [edge_index[1], jnp.full((pad,), N_PAD - 1, jnp.int32)])
    x3 = x.reshape(BATCH_SIZE, NUM_GENES, NUM_CLASS)

    deg0, deg1 = _deg_call()(dst)
    hw3 = _hw_call(x3, exp_emb_table, exp_bias_table,
                   gcn_w.reshape(1, NUM_CLASS))

    dis3, g3 = _mix_call(hw3, _slice3(deg0), _slice3(deg1))

    t0, t1 = _msg_call()(src, dst, g3.reshape(N))

    wgt = _score_call(dis3, g3, _slice3(t0), _slice3(t1),
                      gcn_b.reshape(1, 1))
    out = _pool_call(x3, exp_emb_table, exp_bias_table,
                     wgt.reshape(BATCH_SIZE, 1, NUM_GENES), lin_w,
                     lin_b.reshape(1, NUM_CLASS))
    return out.reshape(BATCH_SIZE, NUM_CLASS)


# speedup vs baseline: 137.0007x; 1.0449x over previous
import functools

import jax
import jax.numpy as jnp
from jax import lax
from jax.experimental import pallas as pl
from jax.experimental.pallas import tpu as pltpu
from jax.experimental.pallas import tpu_sc as plsc

NUM_GENES = 10000
NUM_CLASS = 32
BATCH_SIZE = 10
N = NUM_GENES * BATCH_SIZE
E = 1600000
K = 5000

N_PAD = 100096
SLICE = N_PAD // 16
EROWS = E // 128
ROWS_W = 390
CHR = 13
CH = CHR * 128
ITER = ROWS_W // CHR
TAIL_ROWS = EROWS - 32 * ROWS_W

INT_MIN = -2**31


def _zero_fill(buf, n):
    zero = jnp.zeros((16,), jnp.float32)

    def body(i, _):
        buf[pl.ds(i * 16, 16)] = zero
        return 0

    lax.fori_loop(0, n // 16, body, 0)


def _copy_out(c, s, acc_sh, bounce_v, out0_hbm, out1_hbm):
    pltpu.sync_copy(acc_sh.at[pl.ds(s * SLICE, SLICE)], bounce_v)

    @pl.when(c == 0)
    def _():
        pltpu.sync_copy(bounce_v, out0_hbm.at[pl.ds(s * SLICE, SLICE)])

    @pl.when(c == 1)
    def _():
        pltpu.sync_copy(bounce_v, out1_hbm.at[pl.ds(s * SLICE, SLICE)])


def _sc_hist(dst_hbm, out0_hbm, out1_hbm, idx0_v, idx1_v, tidx_v, ones_v,
             bounce_v, acc_sh, sem0, sem1):
    c = lax.axis_index("c")
    s = lax.axis_index("s")
    wid = s * 2 + c
    base = wid * ROWS_W * 128

    one = jnp.ones((16,), jnp.float32)

    def fill(i, _):
        ones_v[pl.ds(i * 16, 16)] = one
        return 0

    lax.fori_loop(0, CH // 16, fill, 0)

    _zero_fill(bounce_v, SLICE)
    pltpu.sync_copy(bounce_v, acc_sh.at[pl.ds(s * SLICE, SLICE)])
    plsc.subcore_barrier()

    idx = [idx0_v, idx1_v]
    sems = [sem0, sem1]
    loads = [None] * ITER
    for i in range(2):
        loads[i] = pltpu.async_copy(
            dst_hbm.at[pl.ds(base + i * CH, CH)], idx[i], sems[i])
    for i in range(ITER):
        b = i % 2
        loads[i].wait()
        pltpu.sync_copy(ones_v, acc_sh.at[idx[b]], add=True)
        if i + 2 < ITER:
            loads[i + 2] = pltpu.async_copy(
                dst_hbm.at[pl.ds(base + (i + 2) * CH, CH)], idx[b], sems[b])

    @pl.when(wid < TAIL_ROWS)
    def _():
        t0 = (32 * ROWS_W + wid) * 128
        pltpu.sync_copy(dst_hbm.at[pl.ds(t0, 128)], tidx_v)
        pltpu.sync_copy(ones_v.at[pl.ds(0, 128)], acc_sh.at[tidx_v],
                        add=True)

    plsc.subcore_barrier()
    _copy_out(c, s, acc_sh, bounce_v, out0_hbm, out1_hbm)


def _sc_msg(src_hbm, dst_hbm, g_hbm, out0_hbm, out1_hbm,
            sidx0_v, sidx1_v, sidx2_v, didx0_v, didx1_v, didx2_v,
            vals0_v, vals1_v, vals2_v,
            tsidx_v, tdidx_v, tvals_v, g_v, bounce_v, acc_sh,
            sems0, sems1, sems2, semd0, semd1, semd2,
            semc0, semc1, semc2):
    c = lax.axis_index("c")
    s = lax.axis_index("s")
    wid = s * 2 + c
    base = wid * ROWS_W * 128

    _zero_fill(bounce_v, SLICE)
    pltpu.sync_copy(bounce_v, acc_sh.at[pl.ds(s * SLICE, SLICE)])
    pltpu.sync_copy(g_hbm, g_v)
    plsc.subcore_barrier()

    sidx = [sidx0_v, sidx1_v, sidx2_v]
    didx = [didx0_v, didx1_v, didx2_v]
    vals = [vals0_v, vals1_v, vals2_v]
    sems = [sems0, sems1, sems2]
    semd = [semd0, semd1, semd2]
    semc = [semc0, semc1, semc2]
    sload = [None] * ITER
    dload = [None] * ITER
    scat = [None] * ITER

    def gather_chunk(src_idx, dst_vals, n):
        def body(k, _):
            i16 = src_idx[pl.ds(k * 16, 16)]
            dst_vals[pl.ds(k * 16, 16)] = plsc.load_gather(g_v, [i16])
            return 0

        lax.fori_loop(0, n // 16, body, 0)

    def fire(i):
        b = i % 3
        sload[i] = pltpu.async_copy(
            src_hbm.at[pl.ds(base + i * CH, CH)], sidx[b], sems[b])
        dload[i] = pltpu.async_copy(
            dst_hbm.at[pl.ds(base + i * CH, CH)], didx[b], semd[b])

    fire(0)
    fire(1)
    for i in range(ITER):
        b = i % 3
        sload[i].wait()
        gather_chunk(sidx[b], vals[b], CH)
        dload[i].wait()
        scat[i] = pltpu.async_copy(vals[b], acc_sh.at[didx[b]], semc[b],
                                   add=True)
        if i >= 1:
            scat[i - 1].wait()
        if i + 2 < ITER:
            fire(i + 2)
    scat[ITER - 1].wait()

    @pl.when(wid < TAIL_ROWS)
    def _():
        t0 = (32 * ROWS_W + wid) * 128
        pltpu.sync_copy(src_hbm.at[pl.ds(t0, 128)], tsidx_v)
        pltpu.sync_copy(dst_hbm.at[pl.ds(t0, 128)], tdidx_v)
        gather_chunk(tsidx_v, tvals_v, 128)
        pltpu.sync_copy(tvals_v, acc_sh.at[tdidx_v], add=True)

    plsc.subcore_barrier()
    _copy_out(c, s, acc_sh, bounce_v, out0_hbm, out1_hbm)


def _deg_call():
    mesh = plsc.VectorSubcoreMesh(core_axis_name="c", subcore_axis_name="s")
    return functools.partial(
        pl.kernel, _sc_hist, mesh=mesh,
        out_type=[jax.ShapeDtypeStruct((N_PAD,), jnp.float32),
                  jax.ShapeDtypeStruct((N_PAD,), jnp.float32)],
        scratch_types=[
            pltpu.VMEM((CH,), jnp.int32),
            pltpu.VMEM((CH,), jnp.int32),
            pltpu.VMEM((128,), jnp.int32),
            pltpu.VMEM((CH,), jnp.float32),
            pltpu.VMEM((SLICE,), jnp.float32),
            pltpu.VMEM_SHARED((N_PAD,), jnp.float32),
            pltpu.SemaphoreType.DMA,
            pltpu.SemaphoreType.DMA,
        ],
    )()


def _msg_call():
    mesh = plsc.VectorSubcoreMesh(core_axis_name="c", subcore_axis_name="s")
    return functools.partial(
        pl.kernel, _sc_msg, mesh=mesh,
        compiler_params=pltpu.CompilerParams(needs_layout_passes=False),
        out_type=[jax.ShapeDtypeStruct((N_PAD,), jnp.float32),
                  jax.ShapeDtypeStruct((N_PAD,), jnp.float32)],
        scratch_types=(
            [pltpu.VMEM((CH,), jnp.int32)] * 6 +
            [pltpu.VMEM((CH,), jnp.float32)] * 3 +
            [pltpu.VMEM((128,), jnp.int32),
             pltpu.VMEM((128,), jnp.int32),
             pltpu.VMEM((128,), jnp.float32),
             pltpu.VMEM((N,), jnp.float32),
             pltpu.VMEM((SLICE,), jnp.float32),
             pltpu.VMEM_SHARED((N_PAD,), jnp.float32)] +
            [pltpu.SemaphoreType.DMA] * 9
        ),
    )()


ROWS_F = NUM_GENES // 4


def _tc_hw(x_ref, emb_ref, pw_ref, bias_ref, ws_ref, hw_ref):
    z = x_ref[0] * emb_ref[...]
    hw4 = jax.lax.dot(z, pw_ref[...],
                      preferred_element_type=jnp.float32)
    hw_ref[0] = hw4 + bias_ref[...] * ws_ref[0, 0]


def _tc_mix(hw_ref, d0_ref, d1_ref, dis_ref, g_ref):
    deg = 1.0 + d0_ref[:, 0, :] + d1_ref[:, 0, :]
    dis = lax.rsqrt(deg)
    dis_ref[:, 0, :] = dis
    g_ref[:, 0, :] = hw_ref[:, 0, :] * dis


def _tc_score(dis_ref, g_ref, t0_ref, t1_ref, gcnb_ref, wgt_ref):
    t = t0_ref[:, 0, :] + t1_ref[:, 0, :]
    score = dis_ref[:, 0, :] * (t + g_ref[:, 0, :]) + gcnb_ref[0, 0]
    bits = lax.bitcast_convert_type(score, jnp.int32)
    skey = bits ^ ((bits >> 31) & jnp.int32(0x7FFFFFFF))

    int_min = jnp.int32(INT_MIN)
    thr_u = jnp.zeros((BATCH_SIZE, 1), jnp.int32)
    for b in range(31, -1, -1):
        bit = int_min if b == 31 else jnp.int32(1 << b)
        cand = thr_u | bit
        cand_s = cand ^ int_min
        cnt = jnp.sum((skey >= cand_s).astype(jnp.int32), axis=1,
                      keepdims=True)
        thr_u = jnp.where(cnt >= K, cand, thr_u)
    thr_s = thr_u ^ int_min

    wgt_ref[...] = jnp.where(skey >= thr_s, jnp.tanh(score), 0.0)


def _tc_pool(x_ref, emb_ref, bias_ref, wgt_ref, linw_ref, linb_ref, out_ref):
    h = emb_ref[...] * x_ref[0] + bias_ref[...]
    wcol = wgt_ref[0, 0].reshape(NUM_GENES, 1)
    pooled = jnp.sum(h * wcol, axis=0) * jnp.float32(1.0 / K)
    row = jnp.sum(linw_ref[...] * pooled[None, :], axis=1)
    out_ref[0, 0, :] = row + linb_ref[0]


def _hw_call(x2, emb2, pw, bias4, ws):
    return pl.pallas_call(
        _tc_hw,
        grid=(BATCH_SIZE,),
        in_specs=[
            pl.BlockSpec((1, ROWS_F, 128), lambda b: (b, 0, 0)),
            pl.BlockSpec((ROWS_F, 128), lambda b: (0, 0)),
            pl.BlockSpec((128, 4), lambda b: (0, 0)),
            pl.BlockSpec((ROWS_F, 4), lambda b: (0, 0)),
            pl.BlockSpec((1, 1), lambda b: (0, 0)),
        ],
        out_specs=pl.BlockSpec((1, ROWS_F, 4), lambda b: (b, 0, 0)),
        out_shape=jax.ShapeDtypeStruct((BATCH_SIZE, ROWS_F, 4), jnp.float32),
    )(x2, emb2, pw, bias4, ws)


def _mix_call(hw3, d03, d13):
    return pl.pallas_call(
        _tc_mix,
        out_shape=[
            jax.ShapeDtypeStruct((BATCH_SIZE, 1, NUM_GENES), jnp.float32),
            jax.ShapeDtypeStruct((BATCH_SIZE, 1, NUM_GENES), jnp.float32),
        ],
    )(hw3, d03, d13)


def _score_call(dis3, g3, t03, t13, gcnb):
    return pl.pallas_call(
        _tc_score,
        out_shape=jax.ShapeDtypeStruct((BATCH_SIZE, NUM_GENES), jnp.float32),
    )(dis3, g3, t03, t13, gcnb)


def _pool_call(x3, emb, bias, wgt3, linw, linb):
    return pl.pallas_call(
        _tc_pool,
        grid=(BATCH_SIZE,),
        in_specs=[
            pl.BlockSpec((1, NUM_GENES, NUM_CLASS), lambda b: (b, 0, 0)),
            pl.BlockSpec((NUM_GENES, NUM_CLASS), lambda b: (0, 0)),
            pl.BlockSpec((NUM_GENES, 1), lambda b: (0, 0)),
            pl.BlockSpec((1, 1, NUM_GENES), lambda b: (b, 0, 0)),
            pl.BlockSpec((NUM_CLASS, NUM_CLASS), lambda b: (0, 0)),
            pl.BlockSpec((1, NUM_CLASS), lambda b: (0, 0)),
        ],
        out_specs=pl.BlockSpec((1, 1, NUM_CLASS), lambda b: (b, 0, 0)),
        out_shape=jax.ShapeDtypeStruct((BATCH_SIZE, 1, NUM_CLASS),
                                       jnp.float32),
    )(x3, emb, bias, wgt3, linw, linb)


def _slice3(a):
    return a[:N].reshape(BATCH_SIZE, 1, NUM_GENES)


def kernel(x, exp_emb_table, exp_bias_table, gcn_w, gcn_b, lin_w, lin_b,
           edge_index, batch):
    src = edge_index[0]
    dst = edge_index[1]
    x3 = x.reshape(BATCH_SIZE, NUM_GENES, NUM_CLASS)
    x2 = x.reshape(BATCH_SIZE, ROWS_F, 128)
    emb2 = exp_emb_table.reshape(ROWS_F, 128)
    bias4 = exp_bias_table.reshape(ROWS_F, 4)
    wv = gcn_w.reshape(NUM_CLASS)
    pw = (jnp.tile(wv, 4)[:, None] *
          ((jnp.arange(128) // 32)[:, None] == jnp.arange(4)[None, :]))
    ws = jnp.sum(wv).reshape(1, 1)

    deg0, deg1 = _deg_call()(dst)
    hw4 = _hw_call(x2, emb2, pw, bias4, ws)
    hw3 = hw4.reshape(BATCH_SIZE, 1, NUM_GENES)

    dis3, g3 = _mix_call(hw3, _slice3(deg0), _slice3(deg1))

    t0, t1 = _msg_call()(src, dst, g3.reshape(N))

    wgt = _score_call(dis3, g3, _slice3(t0), _slice3(t1),
                      gcn_b.reshape(1, 1))
    out = _pool_call(x3, exp_emb_table, exp_bias_table,
                     wgt.reshape(BATCH_SIZE, 1, NUM_GENES), lin_w,
                     lin_b.reshape(1, NUM_CLASS))
    return out.reshape(BATCH_SIZE, NUM_CLASS)
